# baseline (speedup 1.0000x reference)
import functools

import jax
import jax.numpy as jnp
from jax import lax
from jax.experimental import pallas as pl
from jax.experimental.pallas import tpu as pltpu
from jax.experimental.pallas import tpu_sc as plsc

MAXLEN = 200
EMBED_DIM = 32
BATCH = 4096

NC, NS, L = 2, 16, 16
NW = NC * NS
ROWS_PER_CHUNK = 4
IDX_PER_CHUNK = ROWS_PER_CHUNK * MAXLEN
SPLIT_OFFS = (0, 96)
SPLIT_LENS = (96, 104)
GATHERS = ROWS_PER_CHUNK * len(SPLIT_OFFS)
CHUNKS_PER_W = BATCH // (NW * ROWS_PER_CHUNK)
MAXLEN_PAD = 208


def _body(idx_hbm, table_hbm, pos_hbm, out_hbm, idx_v, rows_v, pos_v, gsems):
    wid = lax.axis_index("s") * NC + lax.axis_index("c")

    pltpu.sync_copy(pos_hbm, pos_v)

    def fire(b, c):
        r0 = c * ROWS_PER_CHUNK
        pltpu.sync_copy(
            idx_hbm.at[pl.ds(r0, ROWS_PER_CHUNK), :],
            idx_v.at[b, :, pl.ds(0, MAXLEN)],
        )

        for r in range(ROWS_PER_CHUNK):
            @pl.loop(0, MAXLEN_PAD // L)
            def _remap(k):
                o = k * L
                i = idx_v[b, r, pl.ds(o, L)]
                f = (i & ~31) | ((i & 7) << 2) | ((i >> 3) & 3)
                idx_v[b, r, pl.ds(o, L)] = f

        for r in range(ROWS_PER_CHUNK):
            for off, ln in zip(SPLIT_OFFS, SPLIT_LENS):
                idx = idx_v.at[b, r, pl.ds(off, ln)]
                dst = rows_v.at[b, r, pl.ds(off, ln), :]
                pltpu.async_copy(table_hbm.at[idx], dst, gsems[b])

    def consume(b, c):
        r0 = c * ROWS_PER_CHUNK
        pltpu.make_async_copy(
            out_hbm.at[pl.ds(0, ROWS_PER_CHUNK), :, :], rows_v.at[b], gsems[b]
        ).wait()

        @pl.loop(0, MAXLEN)
        def _pos(p):
            pv0 = pos_v[p, pl.ds(0, L)]
            pv1 = pos_v[p, pl.ds(L, L)]
            for r in range(ROWS_PER_CHUNK):
                plsc.addupdate(rows_v.at[b, r, p, pl.ds(0, L)], pv0)
                plsc.addupdate(rows_v.at[b, r, p, pl.ds(L, L)], pv1)

        pltpu.sync_copy(
            rows_v.at[b], out_hbm.at[pl.ds(r0, ROWS_PER_CHUNK), :, :]
        )

    c0 = wid * CHUNKS_PER_W
    fire(0, c0)
    fire(1, c0 + 1)

    @pl.loop(0, CHUNKS_PER_W, step=2)
    def _chunk(t):
        consume(0, c0 + t)

        @pl.when(t + 2 < CHUNKS_PER_W)
        def _():
            fire(0, c0 + t + 2)

        consume(1, c0 + t + 1)

        @pl.when(t + 3 < CHUNKS_PER_W)
        def _():
            fire(1, c0 + t + 3)


@jax.jit
def _embed(inputs, token_table, pos_table):
    vocab = token_table.shape[0]
    table_lin = (
        token_table.reshape(vocab // 32, 4, 8, EMBED_DIM)
        .transpose(0, 2, 1, 3)
        .reshape(vocab, EMBED_DIM)
    )
    mesh = plsc.VectorSubcoreMesh(core_axis_name="c", subcore_axis_name="s")
    return pl.kernel(
        _body,
        out_type=jax.ShapeDtypeStruct((BATCH, MAXLEN, EMBED_DIM), jnp.float32),
        mesh=mesh,
        scratch_types=[
            pltpu.VMEM((2, ROWS_PER_CHUNK, MAXLEN_PAD), jnp.int32),
            pltpu.VMEM((2, ROWS_PER_CHUNK, MAXLEN, EMBED_DIM), jnp.float32),
            pltpu.VMEM((MAXLEN, EMBED_DIM), jnp.float32),
            [pltpu.SemaphoreType.DMA, pltpu.SemaphoreType.DMA],
        ],
        compiler_params=pltpu.CompilerParams(use_tc_tiling_on_sc=False),
    )(inputs, table_lin, pos_table)


def kernel(inputs, token_table, pos_table):
    return _embed(inputs, token_table, pos_table)


# speedup vs baseline: 1.1977x; 1.1977x over previous
import functools

import jax
import jax.numpy as jnp
from jax import lax
from jax.experimental import pallas as pl
from jax.experimental.pallas import tpu as pltpu
from jax.experimental.pallas import tpu_sc as plsc

MAXLEN = 200
EMBED_DIM = 32
BATCH = 4096

NC, NS, L = 2, 16, 16
NW = NC * NS
ROWS_PER_W = BATCH // NW
ROWS_PER_CHUNK = 4
IDX_PER_CHUNK = ROWS_PER_CHUNK * MAXLEN
SPLIT_OFFS = (0, 96)
SPLIT_LENS = (96, 104)
NBUF = 3
CHUNKS_PER_W = ROWS_PER_W // ROWS_PER_CHUNK


def _body(idx_hbm, table_hbm, pos_hbm, out_hbm, idx_v, rows_v, pos_v, gsems, osems):
    wid = lax.axis_index("s") * NC + lax.axis_index("c")
    row0 = wid * ROWS_PER_W

    pltpu.sync_copy(idx_hbm.at[pl.ds(row0, ROWS_PER_W), :], idx_v)
    pltpu.sync_copy(pos_hbm, pos_v)

    def fire(b, t):
        for r in range(ROWS_PER_CHUNK):
            for off, ln in zip(SPLIT_OFFS, SPLIT_LENS):
                idx = idx_v.at[t * ROWS_PER_CHUNK + r, pl.ds(off, ln)]
                dst = rows_v.at[b, r, pl.ds(off, ln), :]
                pltpu.async_copy(table_hbm.at[idx], dst, gsems[b])

    def refire(b, t):
        pltpu.make_async_copy(
            rows_v.at[b], out_hbm.at[pl.ds(0, ROWS_PER_CHUNK), :, :], osems[b]
        ).wait()
        fire(b, t)

    def consume(b, t):
        pltpu.make_async_copy(
            out_hbm.at[pl.ds(0, ROWS_PER_CHUNK), :, :], rows_v.at[b], gsems[b]
        ).wait()

        @pl.loop(0, MAXLEN)
        def _pos(p):
            pv0 = pos_v[p, pl.ds(0, L)]
            pv1 = pos_v[p, pl.ds(L, L)]
            for r in range(ROWS_PER_CHUNK):
                plsc.addupdate(rows_v.at[b, r, p, pl.ds(0, L)], pv0)
                plsc.addupdate(rows_v.at[b, r, p, pl.ds(L, L)], pv1)

        pltpu.async_copy(
            rows_v.at[b],
            out_hbm.at[pl.ds(row0 + t * ROWS_PER_CHUNK, ROWS_PER_CHUNK), :, :],
            osems[b],
        )

    for b in range(NBUF):
        fire(b, b)

    @pl.loop(0, CHUNKS_PER_W - 2, step=NBUF)
    def _chunk(t):
        for b in range(NBUF):
            consume(b, t + b)

            @pl.when(t + b + NBUF < CHUNKS_PER_W)
            def _():
                refire(b, t + b + NBUF)

    consume(0, CHUNKS_PER_W - 2)
    consume(1, CHUNKS_PER_W - 1)

    for b in range(NBUF):
        pltpu.make_async_copy(
            rows_v.at[b], out_hbm.at[pl.ds(0, ROWS_PER_CHUNK), :, :], osems[b]
        ).wait()


@jax.jit
def _embed(inputs, token_table, pos_table):
    mesh = plsc.VectorSubcoreMesh(core_axis_name="c", subcore_axis_name="s")
    return pl.kernel(
        _body,
        out_type=jax.ShapeDtypeStruct((BATCH, MAXLEN, EMBED_DIM), jnp.float32),
        mesh=mesh,
        scratch_types=[
            pltpu.VMEM((ROWS_PER_W, MAXLEN), jnp.int32),
            pltpu.VMEM((NBUF, ROWS_PER_CHUNK, MAXLEN, EMBED_DIM), jnp.float32),
            pltpu.VMEM((MAXLEN, EMBED_DIM), jnp.float32),
            [pltpu.SemaphoreType.DMA] * NBUF,
            [pltpu.SemaphoreType.DMA] * NBUF,
        ],
        compiler_params=pltpu.CompilerParams(use_tc_tiling_on_sc=False),
    )(inputs, token_table, pos_table)


def kernel(inputs, token_table, pos_table):
    return _embed(inputs, token_table, pos_table)
